# baseline (speedup 1.0000x reference)
import functools

import jax
import jax.numpy as jnp
from jax import lax
from jax.experimental import pallas as pl
from jax.experimental.pallas import tpu as pltpu
from jax.experimental.pallas import tpu_sc as plsc

_TOPK = 8
_N_EXPERTS = 64
_N_TOKENS = 32768
_N_WORKERS = 32
_STRIP = _N_TOKENS // _N_WORKERS
_HALF = _STRIP // 2
_L = 16

_CONST_CACHE = {}


def _noise_eps_t(shape, dtype):
    key = ("epsT", shape, str(dtype))
    if key not in _CONST_CACHE:
        eps = jax.random.normal(jax.random.key(42), shape, dtype=dtype)
        _CONST_CACHE[key] = eps.T.copy()
    return _CONST_CACHE[key]


def _noisy_body(x_ref, et_ref, parts_ref):
    x = x_ref[...]
    xt = x.T
    eps = et_ref[...]
    sp = jnp.maximum(xt, 0.0) + jnp.log1p(jnp.exp(-jnp.abs(xt)))
    noisy = xt + eps * sp
    for i in range(parts_ref.shape[0]):
        parts_ref[i] = noisy[:, i * _STRIP:(i + 1) * _STRIP]


def _make_noisy_parts(logits, eps_t):
    n_tokens, n_experts = logits.shape
    block = 8192
    return pl.pallas_call(
        _noisy_body,
        grid=(n_tokens // block,),
        in_specs=[
            pl.BlockSpec((block, n_experts), lambda i: (i, 0)),
            pl.BlockSpec((n_experts, block), lambda i: (0, i)),
        ],
        out_specs=pl.BlockSpec((block // _STRIP, n_experts, _STRIP),
                               lambda i: (i, 0, 0)),
        out_shape=jax.ShapeDtypeStruct(
            (_N_WORKERS, n_experts, _STRIP), jnp.float32),
    )(logits, eps_t)


def _sc_router_body(parts_hbm, out_hbm, idx_hbm, in_v, out_v, idx_v):
    wid = lax.axis_index("s") * 2 + lax.axis_index("c")
    lane = lax.iota(jnp.int32, _L)
    neg_inf = jnp.full((_L,), -jnp.inf, dtype=jnp.float32)
    zero = jnp.zeros((_L,), dtype=jnp.float32)
    e_consts = [jnp.full((_L,), e, dtype=jnp.int32)
                for e in range(_N_EXPERTS)]

    pltpu.sync_copy(parts_hbm.at[pl.ds(wid * (_N_EXPERTS * _STRIP),
                                       _N_EXPERTS * _STRIP)], in_v)
    for h in range(_STRIP // _HALF):
        def process_group(tokoff, ltok):
            m_list = []
            a_list = []
            for _ in range(_TOPK):
                vs = [in_v[pl.ds(e * _STRIP + tokoff, _L)]
                      for e in range(_N_EXPERTS)]
                pairs = list(zip(vs, e_consts))
                while len(pairs) > 1:
                    nxt = []
                    for i in range(len(pairs) // 2):
                        vl, il = pairs[2 * i]
                        vr, ir = pairs[2 * i + 1]
                        cond = vr > vl
                        nxt.append((jnp.maximum(vl, vr),
                                    jnp.where(cond, ir, il)))
                    pairs = nxt
                m, a = pairs[0]
                m_list.append(m)
                a_list.append(a)
                plsc.store_scatter(in_v, [a * _STRIP + tokoff + lane],
                                   neg_inf)
            ws = [jnp.exp(m - m_list[0]) for m in m_list]
            total = ws[0]
            for w in ws[1:]:
                total = total + w
            inv = 1.0 / total
            base64 = ltok * _N_EXPERTS
            for j in range(_N_EXPERTS):
                plsc.store_scatter(out_v, [base64 + j], zero)
            base8 = ltok * _TOPK
            for k in range(_TOPK):
                plsc.store_scatter(out_v, [base64 + a_list[k]],
                                   ws[k] * inv)
                plsc.store_scatter(idx_v, [base8 + k], a_list[k])

        def group_body(gg, carry):
            for u in range(2):
                g = 2 * gg + u
                process_group(h * _HALF + g * _L, g * _L + lane)
            return carry

        lax.fori_loop(0, _HALF // (2 * _L), group_body, 0)
        row0 = wid * _STRIP + h * _HALF
        pltpu.sync_copy(out_v,
                        out_hbm.at[pl.ds(row0 * _N_EXPERTS,
                                         _HALF * _N_EXPERTS)])
        pltpu.sync_copy(idx_v,
                        idx_hbm.at[pl.ds(row0 * _TOPK, _HALF * _TOPK)])


_sc_router = functools.partial(
    pl.kernel,
    out_type=[
        jax.ShapeDtypeStruct((_N_TOKENS * _N_EXPERTS,), jnp.float32),
        jax.ShapeDtypeStruct((_N_TOKENS * _TOPK,), jnp.int32),
    ],
    mesh=plsc.VectorSubcoreMesh(core_axis_name="c", subcore_axis_name="s"),
    compiler_params=pltpu.CompilerParams(needs_layout_passes=False),
    scratch_types=[
        pltpu.VMEM((_N_EXPERTS * _STRIP,), jnp.float32),
        pltpu.VMEM((_HALF * _N_EXPERTS,), jnp.float32),
        pltpu.VMEM((_HALF * _TOPK,), jnp.int32),
    ],
)(_sc_router_body)


def kernel(logits):
    n_tokens, n_experts = logits.shape
    eps_t = _noise_eps_t(logits.shape, logits.dtype)
    parts = _make_noisy_parts(logits, eps_t)
    out_flat, idx_flat = _sc_router(jnp.reshape(parts, (-1,)))
    return (jnp.reshape(out_flat, (n_tokens, n_experts)),
            jnp.reshape(idx_flat, (n_tokens, _TOPK)))


# speedup vs baseline: 1.3934x; 1.3934x over previous
import functools

import jax
import jax.numpy as jnp
from jax import lax
from jax.experimental import pallas as pl
from jax.experimental.pallas import tpu as pltpu
from jax.experimental.pallas import tpu_sc as plsc

_TOPK = 8
_N_EXPERTS = 64
_N_TOKENS = 32768
_N_WORKERS = 32
_STRIP = _N_TOKENS // _N_WORKERS
_HALF = _STRIP // 2
_L = 16

_CONST_CACHE = {}


def _noise_eps_t(shape, dtype):
    key = ("epsT", shape, str(dtype))
    if key not in _CONST_CACHE:
        eps = jax.random.normal(jax.random.key(42), shape, dtype=dtype)
        _CONST_CACHE[key] = eps.T.copy()
    return _CONST_CACHE[key]


def _noisy_body(x_ref, et_ref, parts_ref):
    x = x_ref[...]
    xt = x.T
    eps = et_ref[...]
    sp = jnp.maximum(xt, 0.0) + jnp.log1p(jnp.exp(-jnp.abs(xt)))
    noisy = xt + eps * sp
    for i in range(parts_ref.shape[0]):
        parts_ref[i] = noisy[:, i * _STRIP:(i + 1) * _STRIP]


def _make_noisy_parts(logits, eps_t):
    n_tokens, n_experts = logits.shape
    block = 8192
    return pl.pallas_call(
        _noisy_body,
        grid=(n_tokens // block,),
        in_specs=[
            pl.BlockSpec((block, n_experts), lambda i: (i, 0)),
            pl.BlockSpec((n_experts, block), lambda i: (0, i)),
        ],
        out_specs=pl.BlockSpec((block // _STRIP, n_experts, _STRIP),
                               lambda i: (i, 0, 0)),
        out_shape=jax.ShapeDtypeStruct(
            (_N_WORKERS, n_experts, _STRIP), jnp.float32),
    )(logits, eps_t)


def _sc_router_body(parts_hbm, out_hbm, idx_hbm, in_v, out_v, idx_v):
    wid = lax.axis_index("s") * 2 + lax.axis_index("c")
    lane = lax.iota(jnp.int32, _L)
    neg_inf = jnp.full((_L,), -jnp.inf, dtype=jnp.float32)
    zero = jnp.zeros((_L,), dtype=jnp.float32)
    e_consts = [jnp.full((_L,), e, dtype=jnp.int32)
                for e in range(_N_EXPERTS)]

    pltpu.sync_copy(parts_hbm.at[pl.ds(wid * (_N_EXPERTS * _STRIP),
                                       _N_EXPERTS * _STRIP)], in_v)
    for h in range(_STRIP // _HALF):
        def process_group(tokoff, ltok):
            col = tokoff + lane

            def tournament(vals, ids):
                pairs = list(zip(vals, ids))
                while len(pairs) > 1:
                    nxt = []
                    for i in range(len(pairs) // 2):
                        vl, il = pairs[2 * i]
                        vr, ir = pairs[2 * i + 1]
                        cond = vr > vl
                        nxt.append((jnp.maximum(vl, vr),
                                    jnp.where(cond, ir, il)))
                    pairs = nxt
                return pairs[0]

            vs = [in_v[pl.ds(e * _STRIP + tokoff, _L)]
                  for e in range(_N_EXPERTS)]
            gmax = []
            for j in range(8):
                t = vs[8 * j:8 * j + 8]
                while len(t) > 1:
                    t = [jnp.maximum(t[2 * i], t[2 * i + 1])
                         for i in range(len(t) // 2)]
                gmax.append(t[0])
            m_list = []
            a_list = []
            for _ in range(_TOPK):
                m, jstar = tournament(gmax, e_consts[:8])
                base_g = (jstar << 13) + col
                hs = [plsc.load_gather(in_v, [base_g + i * _STRIP])
                      for i in range(8)]
                _, e_in = tournament(hs, e_consts[:8])
                a = (jstar << 3) + e_in
                m_list.append(m)
                a_list.append(a)
                plsc.store_scatter(in_v, [base_g + (e_in << 10)], neg_inf)
                hs = [jnp.where(e_in == i, neg_inf, h)
                      for i, h in enumerate(hs)]
                while len(hs) > 1:
                    hs = [jnp.maximum(hs[2 * i], hs[2 * i + 1])
                          for i in range(len(hs) // 2)]
                gmax = [jnp.where(jstar == j, hs[0], g)
                        for j, g in enumerate(gmax)]
            ws = [jnp.exp(m - m_list[0]) for m in m_list]
            total = ws[0]
            for w in ws[1:]:
                total = total + w
            inv = 1.0 / total
            base64 = ltok * _N_EXPERTS
            for j in range(_N_EXPERTS):
                plsc.store_scatter(out_v, [base64 + j], zero)
            base8 = ltok * _TOPK
            for k in range(_TOPK):
                plsc.store_scatter(out_v, [base64 + a_list[k]],
                                   ws[k] * inv)
                plsc.store_scatter(idx_v, [base8 + k], a_list[k])

        def group_body(g, carry):
            process_group(h * _HALF + g * _L, g * _L + lane)
            return carry

        lax.fori_loop(0, _HALF // _L, group_body, 0)
        row0 = wid * _STRIP + h * _HALF
        pltpu.sync_copy(out_v,
                        out_hbm.at[pl.ds(row0 * _N_EXPERTS,
                                         _HALF * _N_EXPERTS)])
        pltpu.sync_copy(idx_v,
                        idx_hbm.at[pl.ds(row0 * _TOPK, _HALF * _TOPK)])


_sc_router = functools.partial(
    pl.kernel,
    out_type=[
        jax.ShapeDtypeStruct((_N_TOKENS * _N_EXPERTS,), jnp.float32),
        jax.ShapeDtypeStruct((_N_TOKENS * _TOPK,), jnp.int32),
    ],
    mesh=plsc.VectorSubcoreMesh(core_axis_name="c", subcore_axis_name="s"),
    compiler_params=pltpu.CompilerParams(needs_layout_passes=False),
    scratch_types=[
        pltpu.VMEM((_N_EXPERTS * _STRIP,), jnp.float32),
        pltpu.VMEM((_HALF * _N_EXPERTS,), jnp.float32),
        pltpu.VMEM((_HALF * _TOPK,), jnp.int32),
    ],
)(_sc_router_body)


def kernel(logits):
    n_tokens, n_experts = logits.shape
    eps_t = _noise_eps_t(logits.shape, logits.dtype)
    parts = _make_noisy_parts(logits, eps_t)
    out_flat, idx_flat = _sc_router(jnp.reshape(parts, (-1,)))
    return (jnp.reshape(out_flat, (n_tokens, n_experts)),
            jnp.reshape(idx_flat, (n_tokens, _TOPK)))


# speedup vs baseline: 2.2956x; 1.6475x over previous
import jax
import jax.numpy as jnp
from jax.experimental import pallas as pl
from jax.experimental.pallas import tpu as pltpu

_TOPK = 8
_BLOCK_TOKENS = 8192

_CONST_CACHE = {}


def _noise_eps_t(shape, dtype):
    key = ("epsT", shape, str(dtype))
    if key not in _CONST_CACHE:
        eps = jax.random.normal(jax.random.key(42), shape, dtype=dtype)
        _CONST_CACHE[key] = eps.T.copy()
    return _CONST_CACHE[key]


def _router_body(x_ref, et_ref, out_ref, idx_ref):
    x = x_ref[...]
    n_experts = x.shape[1]
    xt = x.T
    eps = et_ref[...]
    sp = jnp.maximum(xt, 0.0) + jnp.log1p(jnp.exp(-jnp.abs(xt)))
    orig = xt + eps * sp
    work = orig
    row_f = jax.lax.broadcasted_iota(jnp.int32, work.shape, 0).astype(
        jnp.float32)
    neg_inf = jnp.float32(-jnp.inf)
    idxs = []
    m0 = None
    for k in range(_TOPK):
        m = jnp.max(work, axis=0, keepdims=True)
        if k == 0:
            m0 = m
        a = jnp.min(jnp.where(work == m, row_f, float(n_experts)), axis=0,
                    keepdims=True)
        idxs.append(a)
        work = jnp.where(row_f == a, neg_inf, work)
    kept = work == neg_inf
    w = jnp.where(kept, jnp.exp(orig - m0), 0.0)
    total = jnp.sum(w, axis=0, keepdims=True)
    out = w * (1.0 / total)
    out_ref[...] = out.T
    idx_ref[...] = jnp.concatenate(idxs, axis=0).astype(jnp.int32).T


def kernel(logits):
    n_tokens, n_experts = logits.shape
    eps_t = _noise_eps_t(logits.shape, logits.dtype)
    block = min(_BLOCK_TOKENS, n_tokens)
    grid = n_tokens // block
    out, idx = pl.pallas_call(
        _router_body,
        grid=(grid,),
        in_specs=[
            pl.BlockSpec((block, n_experts), lambda i: (i, 0)),
            pl.BlockSpec((n_experts, block), lambda i: (0, i)),
        ],
        out_specs=[
            pl.BlockSpec((block, n_experts), lambda i: (i, 0)),
            pl.BlockSpec((block, _TOPK), lambda i: (i, 0)),
        ],
        out_shape=[
            jax.ShapeDtypeStruct((n_tokens, n_experts), jnp.float32),
            jax.ShapeDtypeStruct((n_tokens, _TOPK), jnp.int32),
        ],
    )(logits, eps_t)
    return out, idx


# speedup vs baseline: 9.6661x; 4.2107x over previous
import jax
import jax.numpy as jnp
import numpy as np
from jax.experimental import pallas as pl
from jax.experimental.pallas import tpu as pltpu

_TOPK = 8
_BLOCK_TOKENS = 8192

_CONST_CACHE = {}


def _noise_eps_t(shape, dtype):
    key = ("epsT", shape, str(dtype))
    if key not in _CONST_CACHE:
        with jax.ensure_compile_time_eval():
            eps = jax.random.normal(jax.random.key(42), shape, dtype=dtype)
        _CONST_CACHE[key] = np.asarray(eps).T.copy()
    return _CONST_CACHE[key]


def _router_body(xt_ref, et_ref, out_ref, idx_ref):
    xt = xt_ref[...]
    n_experts = xt.shape[0]
    eps = et_ref[...]
    sp = jnp.maximum(xt, 0.0) + jnp.log1p(jnp.exp(-jnp.abs(xt)))
    orig = xt + eps * sp
    work = orig
    row_f = jax.lax.broadcasted_iota(jnp.int32, work.shape, 0).astype(
        jnp.float32)
    neg_inf = jnp.float32(-jnp.inf)
    idxs = []
    m0 = None
    for k in range(_TOPK):
        m = jnp.max(work, axis=0, keepdims=True)
        if k == 0:
            m0 = m
        a = jnp.min(jnp.where(work == m, row_f, float(n_experts)), axis=0,
                    keepdims=True)
        idxs.append(a)
        work = jnp.where(row_f == a, neg_inf, work)
    kept = work == neg_inf
    w = jnp.where(kept, jnp.exp(orig - m0), 0.0)
    total = jnp.sum(w, axis=0, keepdims=True)
    out_ref[...] = w * (1.0 / total)
    idx_ref[...] = jnp.concatenate(idxs, axis=0).astype(jnp.int32)


def kernel(logits):
    n_tokens, n_experts = logits.shape
    eps_t = _noise_eps_t(logits.shape, logits.dtype)
    block = min(_BLOCK_TOKENS, n_tokens)
    grid = n_tokens // block
    out_t, idx_t = pl.pallas_call(
        _router_body,
        grid=(grid,),
        in_specs=[
            pl.BlockSpec((n_experts, block), lambda i: (0, i)),
            pl.BlockSpec((n_experts, block), lambda i: (0, i)),
        ],
        out_specs=[
            pl.BlockSpec((n_experts, block), lambda i: (0, i)),
            pl.BlockSpec((_TOPK, block), lambda i: (0, i)),
        ],
        out_shape=[
            jax.ShapeDtypeStruct((n_experts, n_tokens), jnp.float32),
            jax.ShapeDtypeStruct((_TOPK, n_tokens), jnp.int32),
        ],
    )(logits.T, eps_t)
    return out_t.T, idx_t.T


# speedup vs baseline: 9.8243x; 1.0164x over previous
import jax
import jax.numpy as jnp
import numpy as np
from jax.experimental import pallas as pl
from jax.experimental.pallas import tpu as pltpu

_TOPK = 8
_BLOCK_TOKENS = 4096

_CONST_CACHE = {}


def _noise_eps_t(shape, dtype):
    key = ("epsT", shape, str(dtype))
    if key not in _CONST_CACHE:
        with jax.ensure_compile_time_eval():
            eps = jax.random.normal(jax.random.key(42), shape, dtype=dtype)
        _CONST_CACHE[key] = np.asarray(eps).T.copy()
    return _CONST_CACHE[key]


def _router_body(xt_ref, et_ref, out_ref, idx_ref):
    xt = xt_ref[...]
    n_experts = xt.shape[0]
    eps = et_ref[...]
    sp = jnp.maximum(xt, 0.0) + jnp.log1p(jnp.exp(-jnp.abs(xt)))
    orig = xt + eps * sp
    work = orig
    row_f = jax.lax.broadcasted_iota(jnp.int32, work.shape, 0).astype(
        jnp.float32)
    neg_inf = jnp.float32(-jnp.inf)
    idxs = []
    m0 = None
    for k in range(_TOPK):
        m = jnp.max(work, axis=0, keepdims=True)
        if k == 0:
            m0 = m
        a = jnp.min(jnp.where(work == m, row_f, float(n_experts)), axis=0,
                    keepdims=True)
        idxs.append(a)
        work = jnp.where(row_f == a, neg_inf, work)
    kept = work == neg_inf
    w = jnp.where(kept, jnp.exp(orig - m0), 0.0)
    total = jnp.sum(w, axis=0, keepdims=True)
    out_ref[...] = w * (1.0 / total)
    idx_ref[...] = jnp.concatenate(idxs, axis=0).astype(jnp.int32)


def kernel(logits):
    n_tokens, n_experts = logits.shape
    eps_t = _noise_eps_t(logits.shape, logits.dtype)
    block = min(_BLOCK_TOKENS, n_tokens)
    grid = n_tokens // block
    out_t, idx_t = pl.pallas_call(
        _router_body,
        grid=(grid,),
        in_specs=[
            pl.BlockSpec((n_experts, block), lambda i: (0, i)),
            pl.BlockSpec((n_experts, block), lambda i: (0, i)),
        ],
        out_specs=[
            pl.BlockSpec((n_experts, block), lambda i: (0, i)),
            pl.BlockSpec((_TOPK, block), lambda i: (0, i)),
        ],
        out_shape=[
            jax.ShapeDtypeStruct((n_experts, n_tokens), jnp.float32),
            jax.ShapeDtypeStruct((_TOPK, n_tokens), jnp.int32),
        ],
    )(logits.T, eps_t)
    return out_t.T, idx_t.T
